# baseline (speedup 1.0000x reference)
import functools

import jax
import jax.numpy as jnp
from jax import lax
from jax.experimental import pallas as pl
from jax.experimental.pallas import tpu as pltpu
from jax.experimental.pallas import tpu_sc as plsc

_VOCAB = 1000000
_EMB_DIM = 64
_KNOW_DIM = 32
_OUT_DIM = _EMB_DIM + _KNOW_DIM
_BATCH = 4096
_HIST = 50

_N = _BATCH * _HIST
_CHUNK = 128
_NW = 32
_PER_W = _N // _NW
_ROWS_W = _PER_W // _CHUNK
_LANES = 128


def _sc_body(x_hbm, xw_hbm, xk_hbm, word_hbm, know_hbm, out_hbm,
             x_v, idxw_v, idxk_v, wpair_v, kquad_v, stage_v,
             sem_w, sem_k):
    nc = 2
    wid = lax.axis_index("s") * nc + lax.axis_index("c")
    base = wid * _PER_W
    pltpu.sync_copy(x_hbm.at[pl.ds(base, _PER_W)], x_v)
    pltpu.sync_copy(xw_hbm.at[pl.ds(base, _PER_W)], idxw_v)
    pltpu.sync_copy(xk_hbm.at[pl.ds(base, _PER_W)], idxk_v)

    def step(j, carry):
        cw = pltpu.async_copy(
            word_hbm.at[idxw_v.at[pl.ds(j * _CHUNK, _CHUNK)]], wpair_v,
            sem_w)
        ck = pltpu.async_copy(
            know_hbm.at[idxk_v.at[pl.ds(j * _CHUNK, _CHUNK)]], kquad_v,
            sem_k)
        cw.wait()
        ck.wait()

        def move(g, c):
            vbase = g * 16
            idxvec = x_v[pl.ds(j * _CHUNK + vbase, 16)]
            for t in range(16):
                idx = idxvec[t]
                h = (idx & 1) * _EMB_DIM
                q = (idx & 3) * _KNOW_DIM
                row = vbase + t
                for k in range(4):
                    stage_v[row, pl.ds(16 * k, 16)] = (
                        wpair_v[row, pl.ds(h + 16 * k, 16)])
                for k in range(2):
                    stage_v[row, pl.ds(_EMB_DIM + 16 * k, 16)] = (
                        kquad_v[row, pl.ds(q + 16 * k, 16)])
            return c

        lax.fori_loop(0, _CHUNK // 16, move, 0)

        pltpu.sync_copy(stage_v,
                        out_hbm.at[pl.ds(base + j * _CHUNK, _CHUNK)])
        return carry

    lax.fori_loop(0, _ROWS_W, step, 0)


@jax.jit
def _lookup(x1d, xw1d, xk1d, word_view, know_view):
    mesh = plsc.VectorSubcoreMesh(core_axis_name="c", subcore_axis_name="s")
    return pl.kernel(
        _sc_body,
        out_type=jax.ShapeDtypeStruct((_N, _OUT_DIM), jnp.float32),
        mesh=mesh,
        scratch_types=[
            pltpu.VMEM((_PER_W,), jnp.int32),
            pltpu.VMEM((_PER_W,), jnp.int32),
            pltpu.VMEM((_PER_W,), jnp.int32),
            pltpu.VMEM((_CHUNK, _LANES), jnp.float32),
            pltpu.VMEM((_CHUNK, _LANES), jnp.float32),
            pltpu.VMEM((_CHUNK, _OUT_DIM), jnp.float32),
            pltpu.SemaphoreType.DMA,
            pltpu.SemaphoreType.DMA,
        ],
    )(x1d, xw1d, xk1d, word_view, know_view)


def kernel(x, word_table, knowledge_table):
    x1d = x.astype(jnp.int32).reshape(_N)
    xw1d = x1d >> 1
    xk1d = x1d >> 2
    word_view = word_table.reshape(_VOCAB // 2, 2 * _EMB_DIM)
    know_view = knowledge_table.reshape(_VOCAB // 4, 4 * _KNOW_DIM)
    out = _lookup(x1d, xw1d, xk1d, word_view, know_view)
    return out.reshape(_BATCH, _HIST, _OUT_DIM)


# speedup vs baseline: 1.0548x; 1.0548x over previous
import functools

import jax
import jax.numpy as jnp
from jax import lax
from jax.experimental import pallas as pl
from jax.experimental.pallas import tpu as pltpu
from jax.experimental.pallas import tpu_sc as plsc

_VOCAB = 1000000
_EMB_DIM = 64
_KNOW_DIM = 32
_OUT_DIM = _EMB_DIM + _KNOW_DIM
_BATCH = 4096
_HIST = 50

_N = _BATCH * _HIST
_NW = 32
_BPW = _BATCH // _NW
_BPC = 2
_CHUNK = _BPC * _HIST
_NCHUNK = _BPW // _BPC
_GCHUNK = _N // _CHUNK
_LANES = 128


def _sc_body(xp_hbm, xw_hbm, xk_hbm, word_hbm, know_hbm, out_hbm,
             xp_v, xw_v, xk_v, wp_a, wp_b, kq_a, kq_b, st_a, st_b,
             sw_a, sw_b, sk_a, sk_b):
    nc = 2
    wid = lax.axis_index("s") * nc + lax.axis_index("c")
    crow0 = wid * _NCHUNK
    pltpu.sync_copy(xp_hbm.at[pl.ds(crow0, _NCHUNK)], xp_v)
    pltpu.sync_copy(xw_hbm.at[pl.ds(crow0, _NCHUNK)], xw_v)
    pltpu.sync_copy(xk_hbm.at[pl.ds(crow0, _NCHUNK)], xk_v)

    bufs = ((wp_a, kq_a, st_a, sw_a, sk_a),
            (wp_b, kq_b, st_b, sw_b, sk_b))

    def fire(c, wp, kq, sw, sk):
        pltpu.async_copy(word_hbm.at[xw_v.at[c]], wp, sw)
        pltpu.async_copy(know_hbm.at[xk_v.at[c]], kq, sk)

    fire(0, wp_a, kq_a, sw_a, sk_a)
    fire(1, wp_b, kq_b, sw_b, sk_b)

    def gloop(g, carry):
        for b in (0, 1):
            wp, kq, st, sw, sk = bufs[b]
            c = 2 * g + b
            pltpu.make_async_copy(word_hbm.at[xw_v.at[c]], wp, sw).wait()
            pltpu.make_async_copy(know_hbm.at[xk_v.at[c]], kq, sk).wait()

            def movegrp(bb, hbase, nrows, xvec):
                for t in range(nrows):
                    row = bb * _HIST + hbase + t
                    idx = xvec[t]
                    h = (idx & 1) * _EMB_DIM
                    q = (idx & 3) * _KNOW_DIM
                    for k in range(4):
                        st[bb, hbase + t, pl.ds(16 * k, 16)] = (
                            wp[row, pl.ds(h + 16 * k, 16)])
                    for k in range(2):
                        st[bb, hbase + t, pl.ds(_EMB_DIM + 16 * k, 16)] = (
                            kq[row, pl.ds(q + 16 * k, 16)])

            for bb in (0, 1):
                def mloop(gg, c2, bb=bb):
                    movegrp(bb, gg * 16, 16,
                            xp_v[c, pl.ds(bb * _HIST + gg * 16, 16)])
                    return c2

                lax.fori_loop(0, _HIST // 16, mloop, 0)
                movegrp(bb, 48, 2, xp_v[c, pl.ds(bb * _HIST + 48, 16)])

            pltpu.sync_copy(
                st, out_hbm.at[pl.ds(wid * _BPW + c * _BPC, _BPC)])

            @pl.when(c + 2 < _NCHUNK)
            def _():
                fire(c + 2, wp, kq, sw, sk)
        return carry

    lax.fori_loop(0, _NCHUNK // 2, gloop, 0)


@jax.jit
def _lookup(xp2, xw2, xk2, word_view, know_view):
    mesh = plsc.VectorSubcoreMesh(core_axis_name="c", subcore_axis_name="s")
    return pl.kernel(
        _sc_body,
        out_type=jax.ShapeDtypeStruct((_BATCH, _HIST, _OUT_DIM),
                                      jnp.float32),
        mesh=mesh,
        scratch_types=[
            pltpu.VMEM((_NCHUNK, _LANES), jnp.int32),
            pltpu.VMEM((_NCHUNK, _LANES), jnp.int32),
            pltpu.VMEM((_NCHUNK, _LANES), jnp.int32),
            pltpu.VMEM((_LANES, _LANES), jnp.float32),
            pltpu.VMEM((_LANES, _LANES), jnp.float32),
            pltpu.VMEM((_LANES, _LANES), jnp.float32),
            pltpu.VMEM((_LANES, _LANES), jnp.float32),
            pltpu.VMEM((_BPC, _HIST, _OUT_DIM), jnp.float32),
            pltpu.VMEM((_BPC, _HIST, _OUT_DIM), jnp.float32),
            pltpu.SemaphoreType.DMA,
            pltpu.SemaphoreType.DMA,
            pltpu.SemaphoreType.DMA,
            pltpu.SemaphoreType.DMA,
        ],
    )(xp2, xw2, xk2, word_view, know_view)


def kernel(x, word_table, knowledge_table):
    x1d = x.astype(jnp.int32).reshape(_N)
    xpad = jnp.pad(x1d.reshape(_GCHUNK, _CHUNK), ((0, 0), (0, _LANES - _CHUNK)),
                   mode="edge")
    word_view = word_table.reshape(_VOCAB // 2, 2 * _EMB_DIM)
    know_view = knowledge_table.reshape(_VOCAB // 4, 4 * _KNOW_DIM)
    return _lookup(xpad, xpad >> 1, xpad >> 2, word_view, know_view)
